# baseline (speedup 1.0000x reference)
import functools

import jax
import jax.numpy as jnp
from jax import lax
from jax.experimental import pallas as pl
from jax.experimental.pallas import tpu as pltpu
from jax.experimental.pallas import tpu_sc as plsc

N = 10000
D = 256
E = 80000

L = 16
NSUB = 16
W = 128
CH = 128
E2 = 81920
NT = E2 // CH // NSUB
ACC_ROWS = 10240
ZR = 8
RPT = ACC_ROWS // NSUB
NRC = RPT // ZR


def _dense_body(x_ref, w2_ref, lw_ref, b_ref, g_ref, base_ref):
    x = x_ref[...]
    scores = jnp.dot(x, w2_ref[...], preferred_element_type=jnp.float32)
    m = jnp.max(scores, axis=0, keepdims=True)
    g_ref[...] = jnp.exp(scores - m)
    base_ref[...] = (
        jnp.dot(x, lw_ref[...], preferred_element_type=jnp.float32) + b_ref[...]
    )


_dense = pl.pallas_call(
    _dense_body,
    out_shape=[
        jax.ShapeDtypeStruct((N, 8), jnp.float32),
        jax.ShapeDtypeStruct((N, D), jnp.float32),
    ],
)


def _sc_body(xlo_hbm, xhi_hbm, g0_hbm, g1_hbm, src0_hbm, dst0_hbm,
             src1_hbm, dst1_hbm, blo_hbm, bhi_hbm, outlo_hbm, outhi_hbm,
             acc, sseg,
             srcb0, srcb1, dstb0, dstb1, scix0, scix1, gsb0, gsb1, xb0, xb1,
             zrow, zvec, sall, ra0, ra1, rp0, rp1,
             csem0, csem1, gsem0, gsem1, rsem0, rsem1, wsem0, wsem1, zsem):
    cid = lax.axis_index("c")
    sid = lax.axis_index("s")

    srcb = (srcb0, srcb1)
    dstb = (dstb0, dstb1)
    scix = (scix0, scix1)
    gsb = (gsb0, gsb1)
    xb = (xb0, xb1)
    ra = (ra0, ra1)
    rp = (rp0, rp1)
    csem = (csem0, csem1)
    gsem = (gsem0, gsem1)
    rsem = (rsem0, rsem1)
    wsem = (wsem0, wsem1)

    zv = jnp.zeros((L,), jnp.float32)
    for r in range(ZR):
        for k in range(W // L):
            zrow[r, pl.ds(k * L, L)] = zv

    def zero_vec_body(k, _):
        zvec[pl.ds(k * L, L)] = zv
        return 0

    lax.fori_loop(0, RPT // L, zero_vec_body, 0)

    def run_core(xh_hbm, base_hbm, out_hbm):
        row_base = sid * RPT

        for rel in range(2):
            g_hbm = (g0_hbm, g1_hbm)[rel]
            src_hbm = (src0_hbm, src1_hbm)[rel]
            dst_hbm = (dst0_hbm, dst1_hbm)[rel]
            prev_hbm = base_hbm if rel == 0 else out_hbm

            def zero_body(i, _):
                pltpu.sync_copy(zrow, acc.at[pl.ds(row_base + i * ZR, ZR)])
                return 0

            lax.fori_loop(0, NRC, zero_body, 0)
            pltpu.sync_copy(zvec, sseg.at[pl.ds(row_base, RPT)])
            plsc.subcore_barrier()

            def issue_ids(j, p):
                eb = (sid * NT + j) * CH
                pltpu.async_copy(src_hbm.at[pl.ds(eb, CH)], srcb[p], csem[p])
                pltpu.async_copy(dst_hbm.at[pl.ds(eb, CH)], dstb[p], csem[p])

            def wait_ids(p):
                pltpu.make_async_copy(
                    src_hbm.at[pl.ds(0, CH)], srcb[p], csem[p]).wait()
                pltpu.make_async_copy(
                    dst_hbm.at[pl.ds(0, CH)], dstb[p], csem[p]).wait()

            def issue_gathers(p):
                pltpu.async_copy(g_hbm.at[srcb[p]], gsb[p], gsem[p])
                pltpu.async_copy(xh_hbm.at[srcb[p]], xb[p], gsem[p])

            def wait_gathers(p):
                pltpu.make_async_copy(g_hbm.at[srcb[p]], gsb[p],
                                      gsem[p]).wait()
                pltpu.make_async_copy(xh_hbm.at[srcb[p]], xb[p],
                                      gsem[p]).wait()

            def process(p):
                def scale_body(i2, _):
                    gv = gsb[p][pl.ds(i2 * L, L)]
                    for r in range(L):
                        w = jnp.full((L,), gv[r], jnp.float32)
                        row = i2 * L + r
                        for k in range(W // L):
                            sl = pl.ds(k * L, L)
                            xb[p][row, sl] = xb[p][row, sl] * w
                    return 0

                lax.fori_loop(0, CH // L, scale_body, 0)
                pltpu.sync_copy(gsb[p], sseg.at[scix[p]], add=True)
                pltpu.sync_copy(xb[p], acc.at[scix[p]], add=True)

            issue_ids(0, 0)
            wait_ids(0)
            issue_gathers(0)
            issue_ids(1, 1)

            def edge_pair(jj, _):
                for b in (0, 1):
                    j = 2 * jj + b
                    p, q = b, 1 - b
                    wait_gathers(p)
                    for k in range(CH // L):
                        sl = pl.ds(k * L, L)
                        scix[p][sl] = dstb[p][sl]

                    if b == 0:
                        wait_ids(q)
                        issue_gathers(q)

                        @pl.when(jj < NT // 2 - 1)
                        def _():
                            issue_ids(j + 2, p)
                    else:
                        @pl.when(jj < NT // 2 - 1)
                        def _():
                            wait_ids(q)
                            issue_gathers(q)
                            issue_ids(j + 2, p)

                    process(p)
                return 0

            lax.fori_loop(0, NT // 2, edge_pair, 0)
            plsc.subcore_barrier()

            pltpu.sync_copy(sseg.at[pl.ds(row_base, RPT)],
                            sall.at[pl.ds(0, RPT)])

            def r_issue(c, p):
                r0 = row_base + c * ZR
                pltpu.async_copy(acc.at[pl.ds(r0, ZR)], ra[p], rsem[p])
                pltpu.async_copy(prev_hbm.at[pl.ds(r0, ZR)], rp[p], rsem[p])

            def r_wait(p):
                pltpu.make_async_copy(
                    acc.at[pl.ds(row_base, ZR)], ra[p], rsem[p]).wait()
                pltpu.make_async_copy(
                    prev_hbm.at[pl.ds(row_base, ZR)], rp[p], rsem[p]).wait()

            def w_issue(c, p):
                r0 = row_base + c * ZR
                pltpu.async_copy(rp[p], out_hbm.at[pl.ds(r0, ZR)], wsem[p])

            def w_wait(p):
                pltpu.make_async_copy(
                    rp[p], out_hbm.at[pl.ds(row_base, ZR)], wsem[p]).wait()

            def rb_body(c, _):
                r0 = row_base + c * ZR

                @pl.when(r0 < N)
                def _():
                    pltpu.sync_copy(acc.at[pl.ds(r0, ZR)], ra0)
                    pltpu.sync_copy(prev_hbm.at[pl.ds(r0, ZR)], rp0)
                    rvec = 1.0 / jnp.maximum(sall[pl.ds(c * ZR, L)], 1e-9)
                    for r in range(ZR):
                        w = jnp.full((L,), rvec[r], jnp.float32)
                        for k in range(W // L):
                            sl = pl.ds(k * L, L)
                            rp0[r, sl] = rp0[r, sl] + w * ra0[r, sl]
                    pltpu.sync_copy(rp0, out_hbm.at[pl.ds(r0, ZR)])

                return 0

            lax.fori_loop(0, NRC, rb_body, 0)

    @pl.when(cid == 0)
    def _():
        run_core(xlo_hbm, blo_hbm, outlo_hbm)

    @pl.when(cid == 1)
    def _():
        run_core(xhi_hbm, bhi_hbm, outhi_hbm)


_sc_agg = functools.partial(
    pl.kernel,
    out_type=[
        jax.ShapeDtypeStruct((N, W), jnp.float32),
        jax.ShapeDtypeStruct((N, W), jnp.float32),
    ],
    mesh=plsc.VectorSubcoreMesh(core_axis_name="c", subcore_axis_name="s"),
    scratch_types=[
        pltpu.VMEM_SHARED((ACC_ROWS, W), jnp.float32),
        pltpu.VMEM_SHARED((ACC_ROWS,), jnp.float32),
        pltpu.VMEM((CH,), jnp.int32),
        pltpu.VMEM((CH,), jnp.int32),
        pltpu.VMEM((CH,), jnp.int32),
        pltpu.VMEM((CH,), jnp.int32),
        pltpu.VMEM((CH,), jnp.int32),
        pltpu.VMEM((CH,), jnp.int32),
        pltpu.VMEM((CH,), jnp.float32),
        pltpu.VMEM((CH,), jnp.float32),
        pltpu.VMEM((CH, W), jnp.float32),
        pltpu.VMEM((CH, W), jnp.float32),
        pltpu.VMEM((ZR, W), jnp.float32),
        pltpu.VMEM((RPT,), jnp.float32),
        pltpu.VMEM((RPT + L,), jnp.float32),
        pltpu.VMEM((ZR, W), jnp.float32),
        pltpu.VMEM((ZR, W), jnp.float32),
        pltpu.VMEM((ZR, W), jnp.float32),
        pltpu.VMEM((ZR, W), jnp.float32),
        pltpu.SemaphoreType.DMA,
        pltpu.SemaphoreType.DMA,
        pltpu.SemaphoreType.DMA,
        pltpu.SemaphoreType.DMA,
        pltpu.SemaphoreType.DMA,
        pltpu.SemaphoreType.DMA,
        pltpu.SemaphoreType.DMA,
        pltpu.SemaphoreType.DMA,
        pltpu.SemaphoreType.DMA,
    ],
)(_sc_body)


@jax.jit
def kernel(x, edge_index_rel0, edge_index_rel1, w_rel0, w_rel1, loop_weight,
           h_bias):
    w2 = jnp.zeros((D, 8), jnp.float32)
    w2 = w2.at[:, 0].set(w_rel0[:D]).at[:, 1].set(w_rel1[:D])
    g8, base = _dense(x, w2, loop_weight, h_bias.reshape(1, D))
    pad_src = jnp.zeros((E2 - E,), jnp.int32)
    pad_dst = jnp.full((E2 - E,), N, jnp.int32)
    outlo, outhi = _sc_agg(
        x[:, :W],
        x[:, W:],
        g8[:, 0],
        g8[:, 1],
        jnp.concatenate([edge_index_rel0[0], pad_src]),
        jnp.concatenate([edge_index_rel0[1], pad_dst]),
        jnp.concatenate([edge_index_rel1[0], pad_src]),
        jnp.concatenate([edge_index_rel1[1], pad_dst]),
        base[:, :W],
        base[:, W:],
    )
    return jnp.concatenate([outlo, outhi], axis=1)


# speedup vs baseline: 1.0928x; 1.0928x over previous
import functools

import jax
import jax.numpy as jnp
from jax import lax
from jax.experimental import pallas as pl
from jax.experimental.pallas import tpu as pltpu
from jax.experimental.pallas import tpu_sc as plsc

N = 10000
D = 256
E = 80000

L = 16
NSUB = 16
W = 128
CH = 128
E2 = 81920
NT = E2 // CH // NSUB
ACC_ROWS = 10240
ZR = 16
RPT = ACC_ROWS // NSUB
NRC = RPT // ZR


def _dense_body(x_ref, w2_ref, lw_ref, b_ref, g_ref, base_ref):
    x = x_ref[...]
    scores = jnp.dot(x, w2_ref[...], preferred_element_type=jnp.float32)
    m = jnp.max(scores, axis=0, keepdims=True)
    g_ref[...] = jnp.exp(scores - m)
    base_ref[...] = (
        jnp.dot(x, lw_ref[...], preferred_element_type=jnp.float32) + b_ref[...]
    )


_dense = pl.pallas_call(
    _dense_body,
    out_shape=[
        jax.ShapeDtypeStruct((N, 8), jnp.float32),
        jax.ShapeDtypeStruct((N, D), jnp.float32),
    ],
)


def _sc_body(xlo_hbm, xhi_hbm, g0_hbm, g1_hbm, src0_hbm, dst0_hbm,
             src1_hbm, dst1_hbm, blo_hbm, bhi_hbm, outlo_hbm, outhi_hbm,
             acc, sseg,
             srcb0, srcb1, dstb0, dstb1, scix0, scix1, gsb0, gsb1, xb0, xb1,
             zrow, zvec, sall, ra0, ra1, rp0, rp1,
             csem0, csem1, gsem0, gsem1, rsem0, rsem1, wsem0, wsem1, zsem):
    cid = lax.axis_index("c")
    sid = lax.axis_index("s")

    srcb = (srcb0, srcb1)
    dstb = (dstb0, dstb1)
    scix = (scix0, scix1)
    gsb = (gsb0, gsb1)
    xb = (xb0, xb1)
    ra = (ra0, ra1)
    rp = (rp0, rp1)
    csem = (csem0, csem1)
    gsem = (gsem0, gsem1)
    rsem = (rsem0, rsem1)
    wsem = (wsem0, wsem1)

    zv = jnp.zeros((L,), jnp.float32)
    for r in range(ZR):
        for k in range(W // L):
            zrow[r, pl.ds(k * L, L)] = zv

    def zero_vec_body(k, _):
        zvec[pl.ds(k * L, L)] = zv
        return 0

    lax.fori_loop(0, RPT // L, zero_vec_body, 0)

    def run_core(xh_hbm, base_hbm, out_hbm):
        row_base = sid * RPT

        for rel in range(2):
            g_hbm = (g0_hbm, g1_hbm)[rel]
            src_hbm = (src0_hbm, src1_hbm)[rel]
            dst_hbm = (dst0_hbm, dst1_hbm)[rel]
            prev_hbm = base_hbm if rel == 0 else out_hbm

            def zero_body(i, _):
                pltpu.sync_copy(zrow, acc.at[pl.ds(row_base + i * ZR, ZR)])
                return 0

            lax.fori_loop(0, NRC, zero_body, 0)
            pltpu.sync_copy(zvec, sseg.at[pl.ds(row_base, RPT)])
            plsc.subcore_barrier()

            def issue_ids(j, p):
                eb = (sid * NT + j) * CH
                pltpu.async_copy(src_hbm.at[pl.ds(eb, CH)], srcb[p], csem[p])
                pltpu.async_copy(dst_hbm.at[pl.ds(eb, CH)], dstb[p], csem[p])

            def wait_ids(p):
                pltpu.make_async_copy(
                    src_hbm.at[pl.ds(0, CH)], srcb[p], csem[p]).wait()
                pltpu.make_async_copy(
                    dst_hbm.at[pl.ds(0, CH)], dstb[p], csem[p]).wait()

            def issue_gathers(p):
                pltpu.async_copy(g_hbm.at[srcb[p]], gsb[p], gsem[p])
                pltpu.async_copy(xh_hbm.at[srcb[p]], xb[p], gsem[p])

            def wait_gathers(p):
                pltpu.make_async_copy(g_hbm.at[srcb[p]], gsb[p],
                                      gsem[p]).wait()
                pltpu.make_async_copy(xh_hbm.at[srcb[p]], xb[p],
                                      gsem[p]).wait()

            def process(p):
                def scale_body(i2, _):
                    gv = gsb[p][pl.ds(i2 * L, L)]
                    for r in range(L):
                        w = jnp.full((L,), gv[r], jnp.float32)
                        row = i2 * L + r
                        for k in range(W // L):
                            sl = pl.ds(k * L, L)
                            xb[p][row, sl] = xb[p][row, sl] * w
                    return 0

                lax.fori_loop(0, CH // L, scale_body, 0)
                pltpu.sync_copy(gsb[p], sseg.at[scix[p]], add=True)
                pltpu.sync_copy(xb[p], acc.at[scix[p]], add=True)

            issue_ids(0, 0)
            wait_ids(0)
            issue_gathers(0)
            issue_ids(1, 1)

            def edge_pair(jj, _):
                for b in (0, 1):
                    j = 2 * jj + b
                    p, q = b, 1 - b
                    wait_gathers(p)
                    for k in range(CH // L):
                        sl = pl.ds(k * L, L)
                        scix[p][sl] = dstb[p][sl]

                    if b == 0:
                        wait_ids(q)
                        issue_gathers(q)

                        @pl.when(jj < NT // 2 - 1)
                        def _():
                            issue_ids(j + 2, p)
                    else:
                        @pl.when(jj < NT // 2 - 1)
                        def _():
                            wait_ids(q)
                            issue_gathers(q)
                            issue_ids(j + 2, p)

                    process(p)
                return 0

            lax.fori_loop(0, NT // 2, edge_pair, 0)
            plsc.subcore_barrier()

            pltpu.sync_copy(sseg.at[pl.ds(row_base, RPT)],
                            sall.at[pl.ds(0, RPT)])

            def r_issue(c, p):
                r0 = row_base + c * ZR
                pltpu.async_copy(acc.at[pl.ds(r0, ZR)], ra[p], rsem[p])
                pltpu.async_copy(prev_hbm.at[pl.ds(r0, ZR)], rp[p], rsem[p])

            def r_wait(p):
                pltpu.make_async_copy(
                    acc.at[pl.ds(row_base, ZR)], ra[p], rsem[p]).wait()
                pltpu.make_async_copy(
                    prev_hbm.at[pl.ds(row_base, ZR)], rp[p], rsem[p]).wait()

            def w_issue(c, p):
                r0 = row_base + c * ZR
                pltpu.async_copy(rp[p], out_hbm.at[pl.ds(r0, ZR)], wsem[p])

            def w_wait(p):
                pltpu.make_async_copy(
                    rp[p], out_hbm.at[pl.ds(row_base, ZR)], wsem[p]).wait()

            def rb_body(c, _):
                r0 = row_base + c * ZR

                @pl.when(r0 < N)
                def _():
                    pltpu.sync_copy(acc.at[pl.ds(r0, ZR)], ra0)
                    pltpu.sync_copy(prev_hbm.at[pl.ds(r0, ZR)], rp0)
                    rvec = 1.0 / jnp.maximum(sall[pl.ds(c * ZR, L)], 1e-9)
                    for r in range(ZR):
                        w = jnp.full((L,), rvec[r], jnp.float32)
                        for k in range(W // L):
                            sl = pl.ds(k * L, L)
                            rp0[r, sl] = rp0[r, sl] + w * ra0[r, sl]
                    pltpu.sync_copy(rp0, out_hbm.at[pl.ds(r0, ZR)])

                return 0

            lax.fori_loop(0, NRC, rb_body, 0)

    @pl.when(cid == 0)
    def _():
        run_core(xlo_hbm, blo_hbm, outlo_hbm)

    @pl.when(cid == 1)
    def _():
        run_core(xhi_hbm, bhi_hbm, outhi_hbm)


_sc_agg = functools.partial(
    pl.kernel,
    out_type=[
        jax.ShapeDtypeStruct((N, W), jnp.float32),
        jax.ShapeDtypeStruct((N, W), jnp.float32),
    ],
    mesh=plsc.VectorSubcoreMesh(core_axis_name="c", subcore_axis_name="s"),
    scratch_types=[
        pltpu.VMEM_SHARED((ACC_ROWS, W), jnp.float32),
        pltpu.VMEM_SHARED((ACC_ROWS,), jnp.float32),
        pltpu.VMEM((CH,), jnp.int32),
        pltpu.VMEM((CH,), jnp.int32),
        pltpu.VMEM((CH,), jnp.int32),
        pltpu.VMEM((CH,), jnp.int32),
        pltpu.VMEM((CH,), jnp.int32),
        pltpu.VMEM((CH,), jnp.int32),
        pltpu.VMEM((CH,), jnp.float32),
        pltpu.VMEM((CH,), jnp.float32),
        pltpu.VMEM((CH, W), jnp.float32),
        pltpu.VMEM((CH, W), jnp.float32),
        pltpu.VMEM((ZR, W), jnp.float32),
        pltpu.VMEM((RPT,), jnp.float32),
        pltpu.VMEM((RPT + L,), jnp.float32),
        pltpu.VMEM((ZR, W), jnp.float32),
        pltpu.VMEM((ZR, W), jnp.float32),
        pltpu.VMEM((ZR, W), jnp.float32),
        pltpu.VMEM((ZR, W), jnp.float32),
        pltpu.SemaphoreType.DMA,
        pltpu.SemaphoreType.DMA,
        pltpu.SemaphoreType.DMA,
        pltpu.SemaphoreType.DMA,
        pltpu.SemaphoreType.DMA,
        pltpu.SemaphoreType.DMA,
        pltpu.SemaphoreType.DMA,
        pltpu.SemaphoreType.DMA,
        pltpu.SemaphoreType.DMA,
    ],
)(_sc_body)


@jax.jit
def kernel(x, edge_index_rel0, edge_index_rel1, w_rel0, w_rel1, loop_weight,
           h_bias):
    w2 = jnp.zeros((D, 8), jnp.float32)
    w2 = w2.at[:, 0].set(w_rel0[:D]).at[:, 1].set(w_rel1[:D])
    g8, base = _dense(x, w2, loop_weight, h_bias.reshape(1, D))
    pad_src = jnp.zeros((E2 - E,), jnp.int32)
    pad_dst = jnp.full((E2 - E,), N, jnp.int32)
    outlo, outhi = _sc_agg(
        x[:, :W],
        x[:, W:],
        g8[:, 0],
        g8[:, 1],
        jnp.concatenate([edge_index_rel0[0], pad_src]),
        jnp.concatenate([edge_index_rel0[1], pad_dst]),
        jnp.concatenate([edge_index_rel1[0], pad_src]),
        jnp.concatenate([edge_index_rel1[1], pad_dst]),
        base[:, :W],
        base[:, W:],
    )
    return jnp.concatenate([outlo, outhi], axis=1)


# speedup vs baseline: 1.0956x; 1.0025x over previous
import functools

import jax
import jax.numpy as jnp
from jax import lax
from jax.experimental import pallas as pl
from jax.experimental.pallas import tpu as pltpu
from jax.experimental.pallas import tpu_sc as plsc

N = 10000
D = 256
E = 80000

L = 16
NSUB = 16
W = 128
CH = 128
E2 = 81920
NT = E2 // CH // NSUB
ACC_ROWS = 10240
ZR = 16
RPT = ACC_ROWS // NSUB
NRC = RPT // ZR


def _dense_body(x_ref, w2_ref, lw_ref, b_ref, g_ref, base_ref):
    x = x_ref[...]
    scores = jnp.dot(x, w2_ref[...], preferred_element_type=jnp.float32)
    m = jnp.max(scores, axis=0, keepdims=True)
    g_ref[...] = jnp.exp(scores - m)
    base_ref[...] = (
        jnp.dot(x, lw_ref[...], preferred_element_type=jnp.float32) + b_ref[...]
    )


_dense = pl.pallas_call(
    _dense_body,
    out_shape=[
        jax.ShapeDtypeStruct((N, 8), jnp.float32),
        jax.ShapeDtypeStruct((N, D), jnp.float32),
    ],
)


def _sc_body(xlo_hbm, xhi_hbm, g0_hbm, g1_hbm, src0_hbm, dst0_hbm,
             src1_hbm, dst1_hbm, blo_hbm, bhi_hbm, outlo_hbm, outhi_hbm,
             acc, sseg,
             srcb0, srcb1, dstb0, dstb1, scix0, scix1, gsb0, gsb1, xb0, xb1,
             zrow, zvec, sall, ra0, ra1, rp0, rp1,
             csem0, csem1, gsem0, gsem1, rsem0, rsem1, wsem0, wsem1, zsem):
    cid = lax.axis_index("c")
    sid = lax.axis_index("s")

    srcb = (srcb0, srcb1)
    dstb = (dstb0, dstb1)
    scix = (scix0, scix1)
    gsb = (gsb0, gsb1)
    xb = (xb0, xb1)
    ra = (ra0, ra1)
    rp = (rp0, rp1)
    csem = (csem0, csem1)
    gsem = (gsem0, gsem1)
    rsem = (rsem0, rsem1)
    wsem = (wsem0, wsem1)

    zv = jnp.zeros((L,), jnp.float32)
    for r in range(ZR):
        for k in range(W // L):
            zrow[r, pl.ds(k * L, L)] = zv

    def zero_vec_body(k, _):
        zvec[pl.ds(k * L, L)] = zv
        return 0

    lax.fori_loop(0, RPT // L, zero_vec_body, 0)

    def run_core(xh_hbm, base_hbm, out_hbm):
        row_base = sid * RPT

        for rel in range(2):
            g_hbm = (g0_hbm, g1_hbm)[rel]
            src_hbm = (src0_hbm, src1_hbm)[rel]
            dst_hbm = (dst0_hbm, dst1_hbm)[rel]
            prev_hbm = base_hbm if rel == 0 else out_hbm

            def zero_body(i, _):
                pltpu.sync_copy(zrow, acc.at[pl.ds(row_base + i * ZR, ZR)])
                return 0

            lax.fori_loop(0, NRC, zero_body, 0)
            pltpu.sync_copy(zvec, sseg.at[pl.ds(row_base, RPT)])
            plsc.subcore_barrier()

            def issue_ids(j, p):
                eb = (sid * NT + j) * CH
                pltpu.async_copy(src_hbm.at[pl.ds(eb, CH)], srcb[p], csem[p])
                pltpu.async_copy(dst_hbm.at[pl.ds(eb, CH)], dstb[p], csem[p])

            def wait_ids(p):
                pltpu.make_async_copy(
                    src_hbm.at[pl.ds(0, CH)], srcb[p], csem[p]).wait()
                pltpu.make_async_copy(
                    dst_hbm.at[pl.ds(0, CH)], dstb[p], csem[p]).wait()

            def issue_gathers(p):
                pltpu.async_copy(g_hbm.at[srcb[p]], gsb[p], gsem[p])
                pltpu.async_copy(xh_hbm.at[srcb[p]], xb[p], gsem[p])

            def wait_gathers(p):
                pltpu.make_async_copy(g_hbm.at[srcb[p]], gsb[p],
                                      gsem[p]).wait()
                pltpu.make_async_copy(xh_hbm.at[srcb[p]], xb[p],
                                      gsem[p]).wait()

            def process(p):
                pltpu.async_copy(gsb[p], sseg.at[scix[p]], zsem, add=True)

                def scale_body(i2, _):
                    gv = gsb[p][pl.ds(i2 * L, L)]
                    for r in range(L):
                        w = jnp.full((L,), gv[r], jnp.float32)
                        row = i2 * L + r
                        for k in range(W // L):
                            sl = pl.ds(k * L, L)
                            xb[p][row, sl] = xb[p][row, sl] * w
                    return 0

                lax.fori_loop(0, CH // L, scale_body, 0)
                pltpu.async_copy(xb[p], acc.at[scix[p]], zsem, add=True)
                pltpu.make_async_copy(gsb[p], sseg.at[scix[p]],
                                      zsem).wait()
                pltpu.make_async_copy(xb[p], acc.at[scix[p]],
                                      zsem).wait()

            issue_ids(0, 0)
            wait_ids(0)
            issue_gathers(0)
            issue_ids(1, 1)

            def edge_pair(jj, _):
                for b in (0, 1):
                    j = 2 * jj + b
                    p, q = b, 1 - b
                    wait_gathers(p)
                    for k in range(CH // L):
                        sl = pl.ds(k * L, L)
                        scix[p][sl] = dstb[p][sl]

                    if b == 0:
                        wait_ids(q)
                        issue_gathers(q)

                        @pl.when(jj < NT // 2 - 1)
                        def _():
                            issue_ids(j + 2, p)
                    else:
                        @pl.when(jj < NT // 2 - 1)
                        def _():
                            wait_ids(q)
                            issue_gathers(q)
                            issue_ids(j + 2, p)

                    process(p)
                return 0

            lax.fori_loop(0, NT // 2, edge_pair, 0)
            plsc.subcore_barrier()

            pltpu.sync_copy(sseg.at[pl.ds(row_base, RPT)],
                            sall.at[pl.ds(0, RPT)])

            def r_issue(c, p):
                r0 = row_base + c * ZR
                pltpu.async_copy(acc.at[pl.ds(r0, ZR)], ra[p], rsem[p])
                pltpu.async_copy(prev_hbm.at[pl.ds(r0, ZR)], rp[p], rsem[p])

            def r_wait(p):
                pltpu.make_async_copy(
                    acc.at[pl.ds(row_base, ZR)], ra[p], rsem[p]).wait()
                pltpu.make_async_copy(
                    prev_hbm.at[pl.ds(row_base, ZR)], rp[p], rsem[p]).wait()

            def w_issue(c, p):
                r0 = row_base + c * ZR
                pltpu.async_copy(rp[p], out_hbm.at[pl.ds(r0, ZR)], wsem[p])

            def w_wait(p):
                pltpu.make_async_copy(
                    rp[p], out_hbm.at[pl.ds(row_base, ZR)], wsem[p]).wait()

            def rb_body(c, _):
                r0 = row_base + c * ZR

                @pl.when(r0 < N)
                def _():
                    pltpu.sync_copy(acc.at[pl.ds(r0, ZR)], ra0)
                    pltpu.sync_copy(prev_hbm.at[pl.ds(r0, ZR)], rp0)
                    rvec = 1.0 / jnp.maximum(sall[pl.ds(c * ZR, L)], 1e-9)
                    for r in range(ZR):
                        w = jnp.full((L,), rvec[r], jnp.float32)
                        for k in range(W // L):
                            sl = pl.ds(k * L, L)
                            rp0[r, sl] = rp0[r, sl] + w * ra0[r, sl]
                    pltpu.sync_copy(rp0, out_hbm.at[pl.ds(r0, ZR)])

                return 0

            lax.fori_loop(0, NRC, rb_body, 0)

    @pl.when(cid == 0)
    def _():
        run_core(xlo_hbm, blo_hbm, outlo_hbm)

    @pl.when(cid == 1)
    def _():
        run_core(xhi_hbm, bhi_hbm, outhi_hbm)


_sc_agg = functools.partial(
    pl.kernel,
    out_type=[
        jax.ShapeDtypeStruct((N, W), jnp.float32),
        jax.ShapeDtypeStruct((N, W), jnp.float32),
    ],
    mesh=plsc.VectorSubcoreMesh(core_axis_name="c", subcore_axis_name="s"),
    scratch_types=[
        pltpu.VMEM_SHARED((ACC_ROWS, W), jnp.float32),
        pltpu.VMEM_SHARED((ACC_ROWS,), jnp.float32),
        pltpu.VMEM((CH,), jnp.int32),
        pltpu.VMEM((CH,), jnp.int32),
        pltpu.VMEM((CH,), jnp.int32),
        pltpu.VMEM((CH,), jnp.int32),
        pltpu.VMEM((CH,), jnp.int32),
        pltpu.VMEM((CH,), jnp.int32),
        pltpu.VMEM((CH,), jnp.float32),
        pltpu.VMEM((CH,), jnp.float32),
        pltpu.VMEM((CH, W), jnp.float32),
        pltpu.VMEM((CH, W), jnp.float32),
        pltpu.VMEM((ZR, W), jnp.float32),
        pltpu.VMEM((RPT,), jnp.float32),
        pltpu.VMEM((RPT + L,), jnp.float32),
        pltpu.VMEM((ZR, W), jnp.float32),
        pltpu.VMEM((ZR, W), jnp.float32),
        pltpu.VMEM((ZR, W), jnp.float32),
        pltpu.VMEM((ZR, W), jnp.float32),
        pltpu.SemaphoreType.DMA,
        pltpu.SemaphoreType.DMA,
        pltpu.SemaphoreType.DMA,
        pltpu.SemaphoreType.DMA,
        pltpu.SemaphoreType.DMA,
        pltpu.SemaphoreType.DMA,
        pltpu.SemaphoreType.DMA,
        pltpu.SemaphoreType.DMA,
        pltpu.SemaphoreType.DMA,
    ],
)(_sc_body)


@jax.jit
def kernel(x, edge_index_rel0, edge_index_rel1, w_rel0, w_rel1, loop_weight,
           h_bias):
    w2 = jnp.zeros((D, 8), jnp.float32)
    w2 = w2.at[:, 0].set(w_rel0[:D]).at[:, 1].set(w_rel1[:D])
    g8, base = _dense(x, w2, loop_weight, h_bias.reshape(1, D))
    pad_src = jnp.zeros((E2 - E,), jnp.int32)
    pad_dst = jnp.full((E2 - E,), N, jnp.int32)
    outlo, outhi = _sc_agg(
        x[:, :W],
        x[:, W:],
        g8[:, 0],
        g8[:, 1],
        jnp.concatenate([edge_index_rel0[0], pad_src]),
        jnp.concatenate([edge_index_rel0[1], pad_dst]),
        jnp.concatenate([edge_index_rel1[0], pad_src]),
        jnp.concatenate([edge_index_rel1[1], pad_dst]),
        base[:, :W],
        base[:, W:],
    )
    return jnp.concatenate([outlo, outhi], axis=1)


# speedup vs baseline: 1.1257x; 1.0275x over previous
import functools

import jax
import jax.numpy as jnp
from jax import lax
from jax.experimental import pallas as pl
from jax.experimental.pallas import tpu as pltpu
from jax.experimental.pallas import tpu_sc as plsc

N = 10000
D = 256
E = 80000

L = 16
NSUB = 16
W = 128
CH = 128
E2 = 81920
NT = E2 // CH // NSUB
ACC_ROWS = 10240
ZR = 16
RPT = ACC_ROWS // NSUB
NRC = RPT // ZR


def _dense_body(x_ref, w2_ref, lw_ref, b_ref, g_ref,
                xg0lo_ref, xg0hi_ref, xg1lo_ref, xg1hi_ref,
                blo_ref, bhi_ref):
    x = x_ref[...]
    scores = jnp.dot(x, w2_ref[...], preferred_element_type=jnp.float32)
    m = jnp.max(scores, axis=0, keepdims=True)
    g = jnp.exp(scores - m)
    g_ref[...] = g
    xg0 = x * g[:, 0:1]
    xg1 = x * g[:, 1:2]
    xg0lo_ref[...] = xg0[:, :W]
    xg0hi_ref[...] = xg0[:, W:]
    xg1lo_ref[...] = xg1[:, :W]
    xg1hi_ref[...] = xg1[:, W:]
    base = jnp.dot(x, lw_ref[...], preferred_element_type=jnp.float32) + b_ref[...]
    blo_ref[...] = base[:, :W]
    bhi_ref[...] = base[:, W:]


_dense = pl.pallas_call(
    _dense_body,
    out_shape=[
        jax.ShapeDtypeStruct((N, 8), jnp.float32),
        jax.ShapeDtypeStruct((N, W), jnp.float32),
        jax.ShapeDtypeStruct((N, W), jnp.float32),
        jax.ShapeDtypeStruct((N, W), jnp.float32),
        jax.ShapeDtypeStruct((N, W), jnp.float32),
        jax.ShapeDtypeStruct((N, W), jnp.float32),
        jax.ShapeDtypeStruct((N, W), jnp.float32),
    ],
)


def _sc_body(xg0lo_hbm, xg0hi_hbm, xg1lo_hbm, xg1hi_hbm,
             g0_hbm, g1_hbm, src0_hbm, dst0_hbm,
             src1_hbm, dst1_hbm, blo_hbm, bhi_hbm, outlo_hbm, outhi_hbm,
             acc, sseg,
             srcb0, srcb1, dstb0, dstb1, scix0, scix1, gsb0, gsb1, xb0, xb1,
             zrow, zvec, sall, ra0, rp0,
             csem0, csem1, gsem0, gsem1, ssem):
    cid = lax.axis_index("c")
    sid = lax.axis_index("s")

    srcb = (srcb0, srcb1)
    dstb = (dstb0, dstb1)
    scix = (scix0, scix1)
    gsb = (gsb0, gsb1)
    xb = (xb0, xb1)
    csem = (csem0, csem1)
    gsem = (gsem0, gsem1)

    zv = jnp.zeros((L,), jnp.float32)
    for r in range(ZR):
        for k in range(W // L):
            zrow[r, pl.ds(k * L, L)] = zv

    def zero_vec_body(k, _):
        zvec[pl.ds(k * L, L)] = zv
        return 0

    lax.fori_loop(0, RPT // L, zero_vec_body, 0)

    def run_core(xg0h_hbm, xg1h_hbm, base_hbm, out_hbm):
        row_base = sid * RPT

        for rel in range(2):
            g_hbm = (g0_hbm, g1_hbm)[rel]
            xgh_hbm = (xg0h_hbm, xg1h_hbm)[rel]
            src_hbm = (src0_hbm, src1_hbm)[rel]
            dst_hbm = (dst0_hbm, dst1_hbm)[rel]
            prev_hbm = base_hbm if rel == 0 else out_hbm

            def zero_body(i, _):
                pltpu.sync_copy(zrow, acc.at[pl.ds(row_base + i * ZR, ZR)])
                return 0

            lax.fori_loop(0, NRC, zero_body, 0)
            pltpu.sync_copy(zvec, sseg.at[pl.ds(row_base, RPT)])
            plsc.subcore_barrier()

            def issue_ids(j, p):
                eb = (sid * NT + j) * CH
                pltpu.async_copy(src_hbm.at[pl.ds(eb, CH)], srcb[p], csem[p])
                pltpu.async_copy(dst_hbm.at[pl.ds(eb, CH)], dstb[p], csem[p])

            def wait_ids(p):
                pltpu.make_async_copy(
                    src_hbm.at[pl.ds(0, CH)], srcb[p], csem[p]).wait()
                pltpu.make_async_copy(
                    dst_hbm.at[pl.ds(0, CH)], dstb[p], csem[p]).wait()

            def issue_gathers(p):
                pltpu.async_copy(g_hbm.at[srcb[p]], gsb[p], gsem[p])
                pltpu.async_copy(xgh_hbm.at[srcb[p]], xb[p], gsem[p])

            def wait_gathers(p):
                pltpu.make_async_copy(g_hbm.at[srcb[p]], gsb[p],
                                      gsem[p]).wait()
                pltpu.make_async_copy(xgh_hbm.at[srcb[p]], xb[p],
                                      gsem[p]).wait()

            def process(p):
                pltpu.async_copy(gsb[p], sseg.at[scix[p]], ssem, add=True)
                pltpu.async_copy(xb[p], acc.at[scix[p]], ssem, add=True)
                pltpu.make_async_copy(gsb[p], sseg.at[scix[p]], ssem).wait()
                pltpu.make_async_copy(xb[p], acc.at[scix[p]], ssem).wait()

            issue_ids(0, 0)
            wait_ids(0)
            issue_gathers(0)
            issue_ids(1, 1)

            def edge_pair(jj, _):
                for b in (0, 1):
                    j = 2 * jj + b
                    p, q = b, 1 - b
                    wait_gathers(p)
                    for k in range(CH // L):
                        sl = pl.ds(k * L, L)
                        scix[p][sl] = dstb[p][sl]

                    if b == 0:
                        wait_ids(q)
                        issue_gathers(q)

                        @pl.when(jj < NT // 2 - 1)
                        def _():
                            issue_ids(j + 2, p)
                    else:
                        @pl.when(jj < NT // 2 - 1)
                        def _():
                            wait_ids(q)
                            issue_gathers(q)
                            issue_ids(j + 2, p)

                    process(p)
                return 0

            lax.fori_loop(0, NT // 2, edge_pair, 0)
            plsc.subcore_barrier()

            pltpu.sync_copy(sseg.at[pl.ds(row_base, RPT)],
                            sall.at[pl.ds(0, RPT)])

            def rb_body(c, _):
                r0 = row_base + c * ZR

                @pl.when(r0 < N)
                def _():
                    pltpu.sync_copy(acc.at[pl.ds(r0, ZR)], ra0)
                    pltpu.sync_copy(prev_hbm.at[pl.ds(r0, ZR)], rp0)
                    rvec = 1.0 / jnp.maximum(sall[pl.ds(c * ZR, L)], 1e-9)
                    for r in range(ZR):
                        w = jnp.full((L,), rvec[r], jnp.float32)
                        for k in range(W // L):
                            sl = pl.ds(k * L, L)
                            rp0[r, sl] = rp0[r, sl] + w * ra0[r, sl]
                    pltpu.sync_copy(rp0, out_hbm.at[pl.ds(r0, ZR)])

                return 0

            lax.fori_loop(0, NRC, rb_body, 0)

    @pl.when(cid == 0)
    def _():
        run_core(xg0lo_hbm, xg1lo_hbm, blo_hbm, outlo_hbm)

    @pl.when(cid == 1)
    def _():
        run_core(xg0hi_hbm, xg1hi_hbm, bhi_hbm, outhi_hbm)


_sc_agg = functools.partial(
    pl.kernel,
    out_type=[
        jax.ShapeDtypeStruct((N, W), jnp.float32),
        jax.ShapeDtypeStruct((N, W), jnp.float32),
    ],
    mesh=plsc.VectorSubcoreMesh(core_axis_name="c", subcore_axis_name="s"),
    scratch_types=[
        pltpu.VMEM_SHARED((ACC_ROWS, W), jnp.float32),
        pltpu.VMEM_SHARED((ACC_ROWS,), jnp.float32),
        pltpu.VMEM((CH,), jnp.int32),
        pltpu.VMEM((CH,), jnp.int32),
        pltpu.VMEM((CH,), jnp.int32),
        pltpu.VMEM((CH,), jnp.int32),
        pltpu.VMEM((CH,), jnp.int32),
        pltpu.VMEM((CH,), jnp.int32),
        pltpu.VMEM((CH,), jnp.float32),
        pltpu.VMEM((CH,), jnp.float32),
        pltpu.VMEM((CH, W), jnp.float32),
        pltpu.VMEM((CH, W), jnp.float32),
        pltpu.VMEM((ZR, W), jnp.float32),
        pltpu.VMEM((RPT,), jnp.float32),
        pltpu.VMEM((RPT + L,), jnp.float32),
        pltpu.VMEM((ZR, W), jnp.float32),
        pltpu.VMEM((ZR, W), jnp.float32),
        pltpu.SemaphoreType.DMA,
        pltpu.SemaphoreType.DMA,
        pltpu.SemaphoreType.DMA,
        pltpu.SemaphoreType.DMA,
        pltpu.SemaphoreType.DMA,
    ],
)(_sc_body)


@jax.jit
def kernel(x, edge_index_rel0, edge_index_rel1, w_rel0, w_rel1, loop_weight,
           h_bias):
    w2 = jnp.zeros((D, 8), jnp.float32)
    w2 = w2.at[:, 0].set(w_rel0[:D]).at[:, 1].set(w_rel1[:D])
    g8, xg0lo, xg0hi, xg1lo, xg1hi, blo, bhi = _dense(
        x, w2, loop_weight, h_bias.reshape(1, D))
    pad_src = jnp.zeros((E2 - E,), jnp.int32)
    pad_dst = jnp.full((E2 - E,), N, jnp.int32)
    src0 = jnp.concatenate([edge_index_rel0[0], pad_src])
    dst0 = jnp.concatenate([edge_index_rel0[1], pad_dst])
    src1 = jnp.concatenate([edge_index_rel1[0], pad_src])
    dst1 = jnp.concatenate([edge_index_rel1[1], pad_dst])
    outlo, outhi = _sc_agg(
        xg0lo, xg0hi, xg1lo, xg1hi,
        g8[:, 0], g8[:, 1], src0, dst0, src1, dst1, blo, bhi,
    )
    return jnp.concatenate([outlo, outhi], axis=1)
